# baseline (speedup 1.0000x reference)
import functools

import jax
import jax.numpy as jnp
from jax.experimental import pallas as pl
from jax.experimental.pallas import tpu as pltpu

_V = 8192
_R = 16


def _body(x_ref, t_ref, *refs):
    row_refs = refs[:_R]
    out_ref, loss_ref = refs[_R], refs[_R + 1]
    i = pl.program_id(0)
    n = pl.num_programs(0)

    stack = jnp.concatenate([r[0] for r in row_refs], axis=0)
    out_ref[...] = stack

    m = jnp.max(stack, axis=1, keepdims=True)
    s = jnp.sum(jnp.exp(stack - m), axis=1, keepdims=True)
    lse = m + jnp.log(s)

    tvals = t_ref[0, 0, :]
    cols = jax.lax.broadcasted_iota(jnp.int32, (_R, _V), 1)
    tgt = jnp.sum(
        jnp.where(cols == tvals[:, None], stack, 0.0), axis=1, keepdims=True
    )
    partial = jnp.sum(lse - tgt)

    @pl.when(i == 0)
    def _():
        loss_ref[0, 0] = 0.0

    loss_ref[0, 0] += partial

    @pl.when(i == n - 1)
    def _():
        loss_ref[0, 0] = loss_ref[0, 0] / (n * _R)


@functools.partial(jax.jit, static_argnames=())
def kernel(x, targets, table):
    B, T = x.shape
    N = B * T
    x_flat = x.reshape(N)
    t_resh = targets.reshape(N // _R, 1, _R)
    grid = N // _R

    table3 = table.reshape(_V, 1, _V)

    def row_spec(k):
        return pl.BlockSpec((1, 1, _V), lambda i, xr, k=k: (xr[i * _R + k], 0, 0))

    grid_spec = pltpu.PrefetchScalarGridSpec(
        num_scalar_prefetch=1,
        grid=(grid,),
        in_specs=[pl.BlockSpec((1, 1, _R), lambda i, xr: (i, 0, 0))]
        + [row_spec(k) for k in range(_R)],
        out_specs=[
            pl.BlockSpec((_R, _V), lambda i, xr: (i, 0)),
            pl.BlockSpec((1, 1), lambda i, xr: (0, 0), memory_space=pltpu.SMEM),
        ],
    )

    logits_flat, loss = pl.pallas_call(
        _body,
        grid_spec=grid_spec,
        out_shape=[
            jax.ShapeDtypeStruct((N, _V), jnp.float32),
            jax.ShapeDtypeStruct((1, 1), jnp.float32),
        ],
    )(x_flat, t_resh, *([table3] * _R))

    return logits_flat.reshape(B, T, _V), loss[0, 0]


# speedup vs baseline: 1.0048x; 1.0048x over previous
import functools

import jax
import jax.numpy as jnp
from jax.experimental import pallas as pl
from jax.experimental.pallas import tpu as pltpu

_V = 8192
_R = 16


def _body(x_ref, t_ref, *refs):
    row_refs = refs[:_R]
    out_ref, loss_ref = refs[_R], refs[_R + 1]
    i = pl.program_id(0)
    n = pl.num_programs(0)

    stack = jnp.concatenate([r[0] for r in row_refs], axis=0)
    out_ref[...] = stack

    m = jnp.max(stack, axis=1, keepdims=True)
    s = jnp.sum(jnp.exp(stack - m), axis=1, keepdims=True)
    partial = jnp.sum(m + jnp.log(s))

    slabs = []
    tmods = []
    for k in range(_R):
        t = t_ref[i * _R + k]
        t_al = pl.multiple_of((t // 128) * 128, 128)
        slabs.append(row_refs[k][0, :, pl.ds(t_al, 128)])
        tmods.append(t - t_al)
    slab = jnp.concatenate(slabs, axis=0)
    tmod = jnp.stack(tmods)[:, None]
    lanes = jax.lax.broadcasted_iota(jnp.int32, (_R, 128), 1)
    tsum = jnp.sum(jnp.where(lanes == tmod, slab, 0.0))

    @pl.when(i == 0)
    def _():
        loss_ref[0] = 0.0

    loss_ref[0] += partial - tsum

    @pl.when(i == n - 1)
    def _():
        loss_ref[0] = loss_ref[0] / (n * _R)


@jax.jit
def kernel(x, targets, table):
    B, T = x.shape
    N = B * T
    x_flat = x.reshape(N)
    t_flat = targets.reshape(N)
    grid = N // _R
    table3 = table.reshape(_V, 1, _V)

    def row_spec(k):
        return pl.BlockSpec(
            (1, 1, _V), lambda i, xr, tr, k=k: (xr[i * _R + k], 0, 0)
        )

    grid_spec = pltpu.PrefetchScalarGridSpec(
        num_scalar_prefetch=2,
        grid=(grid,),
        in_specs=[row_spec(k) for k in range(_R)],
        out_specs=[
            pl.BlockSpec((_R, _V), lambda i, xr, tr: (i, 0)),
            pl.BlockSpec(memory_space=pltpu.SMEM),
        ],
    )

    logits_flat, loss = pl.pallas_call(
        _body,
        grid_spec=grid_spec,
        out_shape=[
            jax.ShapeDtypeStruct((N, _V), jnp.float32),
            jax.ShapeDtypeStruct((1,), jnp.float32),
        ],
    )(x_flat, t_flat, *([table3] * _R))

    return logits_flat.reshape(B, T, _V), loss[0]


# speedup vs baseline: 1.4526x; 1.4457x over previous
import jax
import jax.numpy as jnp
from jax.experimental import pallas as pl
from jax.experimental.pallas import tpu as pltpu

_V = 8192
_R = 16


def _body(x_ref, t_ref, table_ref, out_ref, loss_ref, rows_buf, sems):
    i = pl.program_id(0)
    n = pl.num_programs(0)
    slot = jax.lax.rem(i, 2)
    nxt = jax.lax.rem(i + 1, 2)

    def issue(step, s):
        for k in range(_R):
            r = x_ref[step * _R + k]
            pltpu.make_async_copy(
                table_ref.at[pl.ds(r, 1), :],
                rows_buf.at[s, pl.ds(k, 1), :],
                sems.at[s, k],
            ).start()

    @pl.when(i == 0)
    def _():
        issue(i, slot)

    @pl.when(i + 1 < n)
    def _():
        issue(i + 1, nxt)

    for k in range(_R):
        pltpu.make_async_copy(
            table_ref.at[pl.ds(x_ref[i * _R + k], 1), :],
            rows_buf.at[slot, pl.ds(k, 1), :],
            sems.at[slot, k],
        ).wait()

    rows = rows_buf[slot]
    out_ref[...] = rows

    m = jnp.max(rows, axis=1, keepdims=True)
    s = jnp.sum(jnp.exp(rows - m), axis=1, keepdims=True)
    partial = jnp.sum(m + jnp.log(s))

    slabs = []
    tmods = []
    for k in range(_R):
        t = t_ref[i * _R + k]
        t_al = pl.multiple_of((t // 128) * 128, 128)
        slabs.append(rows_buf[slot, pl.ds(k, 1), pl.ds(t_al, 128)])
        tmods.append(t - t_al)
    slab = jnp.concatenate(slabs, axis=0)
    tmod = jnp.stack(tmods)[:, None]
    lanes = jax.lax.broadcasted_iota(jnp.int32, (_R, 128), 1)
    tsum = jnp.sum(jnp.where(lanes == tmod, slab, 0.0))

    @pl.when(i == 0)
    def _():
        loss_ref[0] = 0.0

    loss_ref[0] += partial - tsum

    @pl.when(i == n - 1)
    def _():
        loss_ref[0] = loss_ref[0] / (n * _R)


@jax.jit
def kernel(x, targets, table):
    B, T = x.shape
    N = B * T
    x_flat = x.reshape(N)
    t_flat = targets.reshape(N)
    grid = N // _R

    grid_spec = pltpu.PrefetchScalarGridSpec(
        num_scalar_prefetch=2,
        grid=(grid,),
        in_specs=[pl.BlockSpec(memory_space=pl.ANY)],
        out_specs=[
            pl.BlockSpec((_R, _V), lambda i, xr, tr: (i, 0)),
            pl.BlockSpec(memory_space=pltpu.SMEM),
        ],
        scratch_shapes=[
            pltpu.VMEM((2, _R, _V), jnp.float32),
            pltpu.SemaphoreType.DMA((2, _R)),
        ],
    )

    logits_flat, loss = pl.pallas_call(
        _body,
        grid_spec=grid_spec,
        out_shape=[
            jax.ShapeDtypeStruct((N, _V), jnp.float32),
            jax.ShapeDtypeStruct((1,), jnp.float32),
        ],
    )(x_flat, t_flat, table)

    return logits_flat.reshape(B, T, _V), loss[0]


# speedup vs baseline: 2.2037x; 1.5170x over previous
import jax
import jax.numpy as jnp
from jax.experimental import pallas as pl
from jax.experimental.pallas import tpu as pltpu

_V = 8192
_R = 32


def _body(x_ref, t_ref, table_ref, out_ref, loss_ref, rows_buf, sems):
    i = pl.program_id(0)
    n = pl.num_programs(0)
    slot = jax.lax.rem(i, 2)
    nxt = jax.lax.rem(i + 1, 2)

    def issue(step, s):
        for k in range(_R):
            r = x_ref[step * _R + k]
            pltpu.make_async_copy(
                table_ref.at[pl.ds(r, 1), :],
                rows_buf.at[s, pl.ds(k, 1), :],
                sems.at[s, k],
            ).start()

    @pl.when(i == 0)
    def _():
        issue(i, slot)

    @pl.when(i + 1 < n)
    def _():
        issue(i + 1, nxt)

    for k in range(_R):
        pltpu.make_async_copy(
            table_ref.at[pl.ds(x_ref[i * _R + k], 1), :],
            rows_buf.at[slot, pl.ds(k, 1), :],
            sems.at[slot, k],
        ).wait()

    rows = rows_buf[slot]
    out_ref[...] = rows

    m = jnp.max(rows, axis=1, keepdims=True)
    s = jnp.sum(jnp.exp(rows - m), axis=1, keepdims=True)
    partial = jnp.sum(m + jnp.log(s))

    slabs = []
    tmods = []
    for k in range(_R):
        t = t_ref[i * _R + k]
        t_al = pl.multiple_of((t // 128) * 128, 128)
        slabs.append(rows_buf[slot, pl.ds(k, 1), pl.ds(t_al, 128)])
        tmods.append(t - t_al)
    slab = jnp.concatenate(slabs, axis=0)
    tmod = jnp.stack(tmods)[:, None]
    lanes = jax.lax.broadcasted_iota(jnp.int32, (_R, 128), 1)
    tsum = jnp.sum(jnp.where(lanes == tmod, slab, 0.0))

    @pl.when(i == 0)
    def _():
        loss_ref[0] = 0.0

    loss_ref[0] += partial - tsum

    @pl.when(i == n - 1)
    def _():
        loss_ref[0] = loss_ref[0] / (n * _R)


@jax.jit
def kernel(x, targets, table):
    B, T = x.shape
    N = B * T
    x_flat = x.reshape(N)
    t_flat = targets.reshape(N)
    grid = N // _R

    grid_spec = pltpu.PrefetchScalarGridSpec(
        num_scalar_prefetch=2,
        grid=(grid,),
        in_specs=[pl.BlockSpec(memory_space=pl.ANY)],
        out_specs=[
            pl.BlockSpec((_R, _V), lambda i, xr, tr: (i, 0)),
            pl.BlockSpec(memory_space=pltpu.SMEM),
        ],
        scratch_shapes=[
            pltpu.VMEM((2, _R, _V), jnp.float32),
            pltpu.SemaphoreType.DMA((2, _R)),
        ],
    )

    logits_flat, loss = pl.pallas_call(
        _body,
        grid_spec=grid_spec,
        out_shape=[
            jax.ShapeDtypeStruct((N, _V), jnp.float32),
            jax.ShapeDtypeStruct((1,), jnp.float32),
        ],
    )(x_flat, t_flat, table)

    return logits_flat.reshape(B, T, _V), loss[0]


# speedup vs baseline: 2.9557x; 1.3413x over previous
import jax
import jax.numpy as jnp
from jax.experimental import pallas as pl
from jax.experimental.pallas import tpu as pltpu

_V = 8192
_R = 64


def _body(x_ref, t_ref, table_ref, out_ref, loss_ref, rows_buf, sems):
    i = pl.program_id(0)
    n = pl.num_programs(0)
    slot = jax.lax.rem(i, 2)
    nxt = jax.lax.rem(i + 1, 2)

    def issue(step, s):
        for k in range(_R):
            r = x_ref[step * _R + k]
            pltpu.make_async_copy(
                table_ref.at[pl.ds(r, 1), :],
                rows_buf.at[s, pl.ds(k, 1), :],
                sems.at[s, k],
            ).start()

    @pl.when(i == 0)
    def _():
        issue(i, slot)

    @pl.when(i + 1 < n)
    def _():
        issue(i + 1, nxt)

    for k in range(_R):
        pltpu.make_async_copy(
            table_ref.at[pl.ds(x_ref[i * _R + k], 1), :],
            rows_buf.at[slot, pl.ds(k, 1), :],
            sems.at[slot, k],
        ).wait()

    rows = rows_buf[slot]
    out_ref[...] = rows

    m = jnp.max(rows, axis=1, keepdims=True)
    s = jnp.sum(jnp.exp(rows - m), axis=1, keepdims=True)
    partial = jnp.sum(m + jnp.log(s))

    slabs = []
    tmods = []
    for k in range(_R):
        t = t_ref[i * _R + k]
        t_al = pl.multiple_of((t // 128) * 128, 128)
        slabs.append(rows_buf[slot, pl.ds(k, 1), pl.ds(t_al, 128)])
        tmods.append(t - t_al)
    slab = jnp.concatenate(slabs, axis=0)
    tmod = jnp.stack(tmods)[:, None]
    lanes = jax.lax.broadcasted_iota(jnp.int32, (_R, 128), 1)
    tsum = jnp.sum(jnp.where(lanes == tmod, slab, 0.0))

    @pl.when(i == 0)
    def _():
        loss_ref[0] = 0.0

    loss_ref[0] += partial - tsum

    @pl.when(i == n - 1)
    def _():
        loss_ref[0] = loss_ref[0] / (n * _R)


@jax.jit
def kernel(x, targets, table):
    B, T = x.shape
    N = B * T
    x_flat = x.reshape(N)
    t_flat = targets.reshape(N)
    grid = N // _R

    grid_spec = pltpu.PrefetchScalarGridSpec(
        num_scalar_prefetch=2,
        grid=(grid,),
        in_specs=[pl.BlockSpec(memory_space=pl.ANY)],
        out_specs=[
            pl.BlockSpec((_R, _V), lambda i, xr, tr: (i, 0)),
            pl.BlockSpec(memory_space=pltpu.SMEM),
        ],
        scratch_shapes=[
            pltpu.VMEM((2, _R, _V), jnp.float32),
            pltpu.SemaphoreType.DMA((2, _R)),
        ],
    )

    logits_flat, loss = pl.pallas_call(
        _body,
        grid_spec=grid_spec,
        out_shape=[
            jax.ShapeDtypeStruct((N, _V), jnp.float32),
            jax.ShapeDtypeStruct((1,), jnp.float32),
        ],
    )(x_flat, t_flat, table)

    return logits_flat.reshape(B, T, _V), loss[0]


# speedup vs baseline: 3.5169x; 1.1899x over previous
import jax
import jax.numpy as jnp
from jax.experimental import pallas as pl
from jax.experimental.pallas import tpu as pltpu

_V = 8192
_R = 128


def _body(x_ref, t_ref, table_ref, out_ref, loss_ref, rows_buf, sems):
    i = pl.program_id(0)
    n = pl.num_programs(0)
    slot = jax.lax.rem(i, 2)
    nxt = jax.lax.rem(i + 1, 2)

    def issue(step, s):
        for k in range(_R):
            r = x_ref[step * _R + k]
            pltpu.make_async_copy(
                table_ref.at[pl.ds(r, 1), :],
                rows_buf.at[s, pl.ds(k, 1), :],
                sems.at[s, k],
            ).start()

    @pl.when(i == 0)
    def _():
        issue(i, slot)

    @pl.when(i + 1 < n)
    def _():
        issue(i + 1, nxt)

    for k in range(_R):
        pltpu.make_async_copy(
            table_ref.at[pl.ds(x_ref[i * _R + k], 1), :],
            rows_buf.at[slot, pl.ds(k, 1), :],
            sems.at[slot, k],
        ).wait()

    rows = rows_buf[slot]
    out_ref[...] = rows

    m = jnp.max(rows, axis=1, keepdims=True)
    s = jnp.sum(jnp.exp(rows - m), axis=1, keepdims=True)
    partial = jnp.sum(m + jnp.log(s))

    slabs = []
    tmods = []
    for k in range(_R):
        t = t_ref[i * _R + k]
        t_al = pl.multiple_of((t // 128) * 128, 128)
        slabs.append(rows_buf[slot, pl.ds(k, 1), pl.ds(t_al, 128)])
        tmods.append(t - t_al)
    slab = jnp.concatenate(slabs, axis=0)
    tmod = jnp.stack(tmods)[:, None]
    lanes = jax.lax.broadcasted_iota(jnp.int32, (_R, 128), 1)
    tsum = jnp.sum(jnp.where(lanes == tmod, slab, 0.0))

    @pl.when(i == 0)
    def _():
        loss_ref[0] = 0.0

    loss_ref[0] += partial - tsum

    @pl.when(i == n - 1)
    def _():
        loss_ref[0] = loss_ref[0] / (n * _R)


@jax.jit
def kernel(x, targets, table):
    B, T = x.shape
    N = B * T
    x_flat = x.reshape(N)
    t_flat = targets.reshape(N)
    grid = N // _R

    grid_spec = pltpu.PrefetchScalarGridSpec(
        num_scalar_prefetch=2,
        grid=(grid,),
        in_specs=[pl.BlockSpec(memory_space=pl.ANY)],
        out_specs=[
            pl.BlockSpec((_R, _V), lambda i, xr, tr: (i, 0)),
            pl.BlockSpec(memory_space=pltpu.SMEM),
        ],
        scratch_shapes=[
            pltpu.VMEM((2, _R, _V), jnp.float32),
            pltpu.SemaphoreType.DMA((2, _R)),
        ],
    )

    logits_flat, loss = pl.pallas_call(
        _body,
        grid_spec=grid_spec,
        out_shape=[
            jax.ShapeDtypeStruct((N, _V), jnp.float32),
            jax.ShapeDtypeStruct((1,), jnp.float32),
        ],
    )(x_flat, t_flat, table)

    return logits_flat.reshape(B, T, _V), loss[0]


# speedup vs baseline: 3.8034x; 1.0815x over previous
import jax
import jax.numpy as jnp
from jax.experimental import pallas as pl
from jax.experimental.pallas import tpu as pltpu

_V = 8192
_R = 256


def _body(x_ref, t_ref, table_ref, out_ref, loss_ref, rows_buf, sems):
    i = pl.program_id(0)
    n = pl.num_programs(0)
    slot = jax.lax.rem(i, 2)
    nxt = jax.lax.rem(i + 1, 2)

    def issue(step, s):
        for k in range(_R):
            r = x_ref[step * _R + k]
            pltpu.make_async_copy(
                table_ref.at[pl.ds(r, 1), :],
                rows_buf.at[s, pl.ds(k, 1), :],
                sems.at[s, k],
            ).start()

    @pl.when(i == 0)
    def _():
        issue(i, slot)

    @pl.when(i + 1 < n)
    def _():
        issue(i + 1, nxt)

    for k in range(_R):
        pltpu.make_async_copy(
            table_ref.at[pl.ds(x_ref[i * _R + k], 1), :],
            rows_buf.at[slot, pl.ds(k, 1), :],
            sems.at[slot, k],
        ).wait()

    rows = rows_buf[slot]
    out_ref[...] = rows

    m = jnp.max(rows, axis=1, keepdims=True)
    s = jnp.sum(jnp.exp(rows - m), axis=1, keepdims=True)
    partial = jnp.sum(m + jnp.log(s))

    slabs = []
    tmods = []
    for k in range(_R):
        t = t_ref[i * _R + k]
        t_al = pl.multiple_of((t // 128) * 128, 128)
        slabs.append(rows_buf[slot, pl.ds(k, 1), pl.ds(t_al, 128)])
        tmods.append(t - t_al)
    slab = jnp.concatenate(slabs, axis=0)
    tmod = jnp.stack(tmods)[:, None]
    lanes = jax.lax.broadcasted_iota(jnp.int32, (_R, 128), 1)
    tsum = jnp.sum(jnp.where(lanes == tmod, slab, 0.0))

    @pl.when(i == 0)
    def _():
        loss_ref[0] = 0.0

    loss_ref[0] += partial - tsum

    @pl.when(i == n - 1)
    def _():
        loss_ref[0] = loss_ref[0] / (n * _R)


@jax.jit
def kernel(x, targets, table):
    B, T = x.shape
    N = B * T
    x_flat = x.reshape(N)
    t_flat = targets.reshape(N)
    grid = N // _R

    grid_spec = pltpu.PrefetchScalarGridSpec(
        num_scalar_prefetch=2,
        grid=(grid,),
        in_specs=[pl.BlockSpec(memory_space=pl.ANY)],
        out_specs=[
            pl.BlockSpec((_R, _V), lambda i, xr, tr: (i, 0)),
            pl.BlockSpec(memory_space=pltpu.SMEM),
        ],
        scratch_shapes=[
            pltpu.VMEM((2, _R, _V), jnp.float32),
            pltpu.SemaphoreType.DMA((2, _R)),
        ],
    )

    logits_flat, loss = pl.pallas_call(
        _body,
        grid_spec=grid_spec,
        out_shape=[
            jax.ShapeDtypeStruct((N, _V), jnp.float32),
            jax.ShapeDtypeStruct((1,), jnp.float32),
        ],
    )(x_flat, t_flat, table)

    return logits_flat.reshape(B, T, _V), loss[0]
